# baseline (speedup 1.0000x reference)
import functools

import jax
import jax.numpy as jnp
from jax import lax
from jax.experimental import pallas as pl
from jax.experimental.pallas import tpu as pltpu
from jax.experimental.pallas import tpu_sc as plsc

_N = 10000
_NPAD = 10240
_E = 320000
_H = 128
_G = 100
_NPG = 100
_V = 32
_OUT = 6

_LANES = 16
_NTILES = 16
_NCORES = 2


def _deg_dinv(dst):
    ept = _E // _NTILES
    chunk = 2000
    colw = _NPAD // _NTILES
    mesh = plsc.VectorSubcoreMesh(core_axis_name="c", subcore_axis_name="s")

    @functools.partial(
        pl.kernel,
        out_type=jax.ShapeDtypeStruct((_NPAD,), jnp.float32),
        mesh=mesh,
        compiler_params=pltpu.CompilerParams(needs_layout_passes=False),
        scratch_types=[
            pltpu.VMEM((chunk,), jnp.int32),
            pltpu.VMEM((_NPAD,), jnp.float32),
            pltpu.VMEM((colw,), jnp.float32),
            pltpu.VMEM_SHARED((_NTILES, _NPAD), jnp.float32),
        ],
    )
    def run(dst_hbm, dinv_hbm, idx_v, hist_v, col_v, slab):
        c = lax.axis_index("c")
        s = lax.axis_index("s")
        zeros16 = jnp.zeros((_LANES,), jnp.float32)
        ones16 = jnp.ones((_LANES,), jnp.float32)

        @pl.when(c == 0)
        def _():
            def zero_hist(i, _):
                hist_v[pl.ds(i * _LANES, _LANES)] = zeros16
                return 0
            lax.fori_loop(0, _NPAD // _LANES, zero_hist, 0)

            def outer(ci, _):
                pltpu.sync_copy(dst_hbm.at[pl.ds(s * ept + ci * chunk, chunk)],
                                idx_v)
                def inner(j, _):
                    iv = idx_v[pl.ds(j * _LANES, _LANES)]
                    plsc.addupdate_scatter(hist_v, [iv], ones16)
                    return 0
                lax.fori_loop(0, chunk // _LANES, inner, 0)
                return 0
            lax.fori_loop(0, ept // chunk, outer, 0)
            pltpu.sync_copy(hist_v, slab.at[s])

        plsc.subcore_barrier()

        @pl.when(c == 0)
        def _():
            def init_col(i, _):
                col_v[pl.ds(i * _LANES, _LANES)] = ones16
                return 0
            lax.fori_loop(0, colw // _LANES, init_col, 0)

            def add_row(k, _):
                pltpu.sync_copy(slab.at[k, pl.ds(s * colw, colw)],
                                hist_v.at[pl.ds(0, colw)])
                def accum(i, _):
                    sl = pl.ds(i * _LANES, _LANES)
                    col_v[sl] = col_v[sl] + hist_v[sl]
                    return 0
                lax.fori_loop(0, colw // _LANES, accum, 0)
                return 0
            lax.fori_loop(0, _NTILES, add_row, 0)

            def rsqrt_newton(i, _):
                sl = pl.ds(i * _LANES, _LANES)
                x = col_v[sl]
                xi = plsc.bitcast(x, jnp.int32)
                yi = jnp.int32(0x5F3759DF) - lax.shift_right_logical(xi, 1)
                y = plsc.bitcast(yi, jnp.float32)
                hx = x * 0.5
                y = y * (1.5 - hx * y * y)
                y = y * (1.5 - hx * y * y)
                y = y * (1.5 - hx * y * y)
                y = y * (1.5 - hx * y * y)
                col_v[sl] = y
                return 0
            lax.fori_loop(0, colw // _LANES, rsqrt_newton, 0)
            pltpu.sync_copy(col_v, dinv_hbm.at[pl.ds(s * colw, colw)])

    return run(dst)


def _edge_scatter(m_s, src2d, dst2d, zrows):
    k_batch = src2d.shape[2]
    rows_per_tile = src2d.shape[0] // (_NTILES * _NCORES)
    accw = _NPAD // _NTILES
    nbuf = 3
    nouter = -(-(rows_per_tile + 1) // nbuf)
    mesh = plsc.VectorSubcoreMesh(core_axis_name="c", subcore_axis_name="s")

    scratch = [pltpu.VMEM_SHARED((_NPAD, _H), jnp.float32)]
    scratch += [pltpu.VMEM((k_batch, _H), jnp.float32) for _ in range(nbuf)]
    scratch += [pltpu.VMEM((1, k_batch), jnp.int32) for _ in range(2 * nbuf)]
    scratch += [pltpu.SemaphoreType.DMA for _ in range(3 * nbuf)]

    @functools.partial(
        pl.kernel,
        out_type=jax.ShapeDtypeStruct((_NCORES * _NPAD, _H), jnp.float32),
        mesh=mesh,
        compiler_params=pltpu.CompilerParams(needs_layout_passes=False),
        scratch_types=scratch,
    )
    def run(m_hbm, src_hbm, dst_hbm, z_hbm, out_hbm, acc, *rest):
        bufs = rest[:nbuf]
        si = rest[nbuf:2 * nbuf]
        di = rest[2 * nbuf:3 * nbuf]
        isem = rest[3 * nbuf:4 * nbuf]
        gsem = rest[4 * nbuf:5 * nbuf]
        ssem = rest[5 * nbuf:6 * nbuf]
        c = lax.axis_index("c")
        s = lax.axis_index("s")
        pltpu.sync_copy(z_hbm.at[pl.ds(s * accw, accw)],
                        acc.at[pl.ds(s * accw, accw)])
        base = (c * _NTILES + s) * rows_per_tile
        pltpu.async_copy(src_hbm.at[base], si[0], isem[0])
        pltpu.async_copy(dst_hbm.at[base], di[0], isem[0])
        plsc.subcore_barrier()

        def wait_idx(q):
            pltpu.make_async_copy(src_hbm.at[base], si[q],
                                  isem[q]).wait()
            pltpu.make_async_copy(dst_hbm.at[base], di[q],
                                  isem[q]).wait()

        def step(i, _):
            for j in range(nbuf):
                b = i * nbuf + j
                jn = (j + 1) % nbuf
                jp = (j - 1) % nbuf
                @pl.when(b + 1 < rows_per_tile)
                def _():
                    @pl.when(b >= 2)
                    def _():
                        pltpu.make_async_copy(
                            bufs[jn], acc.at[di[jn].at[0]], ssem[jn]).wait()
                    pltpu.async_copy(src_hbm.at[base + b + 1], si[jn],
                                     isem[jn])
                    pltpu.async_copy(dst_hbm.at[base + b + 1], di[jn],
                                     isem[jn])
                @pl.when(b < rows_per_tile)
                def _():
                    wait_idx(j)
                    pltpu.async_copy(m_hbm.at[si[j].at[0]], bufs[j], gsem[j])
                @pl.when(jnp.logical_and(b >= 1, b - 1 < rows_per_tile))
                def _():
                    pltpu.make_async_copy(
                        m_hbm.at[si[jp].at[0]], bufs[jp], gsem[jp]).wait()
                    pltpu.async_copy(bufs[jp], acc.at[di[jp].at[0]], ssem[jp],
                                     add=True)
            return 0
        lax.fori_loop(0, nouter, step, 0)

        for j in range(nbuf):
            pltpu.make_async_copy(bufs[j], acc.at[di[j].at[0]],
                                  ssem[j]).wait()

        plsc.subcore_barrier()
        pltpu.sync_copy(acc.at[pl.ds(s * accw, accw)],
                        out_hbm.at[pl.ds(c * _NPAD + s * accw, accw)])

    return run(m_s, src2d, dst2d, zrows)


def _tc_h2(x_pad, w_emb, b_emb2d, w_gcn):
    rows = 1280

    def body(x_ref, we_ref, be_ref, wg_ref, o_ref):
        h = jnp.dot(x_ref[...], we_ref[...],
                    preferred_element_type=jnp.float32) + be_ref[...]
        o_ref[...] = jnp.dot(h, wg_ref[...],
                             preferred_element_type=jnp.float32)

    return pl.pallas_call(
        body,
        grid=(_NPAD // rows,),
        in_specs=[
            pl.BlockSpec((rows, _H), lambda i: (i, 0)),
            pl.BlockSpec((_H, _H), lambda i: (0, 0)),
            pl.BlockSpec((1, _H), lambda i: (0, 0)),
            pl.BlockSpec((_H, _H), lambda i: (0, 0)),
        ],
        out_specs=pl.BlockSpec((rows, _H), lambda i: (i, 0)),
        out_shape=jax.ShapeDtypeStruct((_NPAD, _H), jnp.float32),
    )(x_pad, w_emb, b_emb2d, w_gcn)


def _tc_scale(h2, dinv2d):
    rows = 5120

    def body(h_ref, dv_ref, o_ref):
        o_ref[...] = h_ref[...] * dv_ref[...]

    return pl.pallas_call(
        body,
        grid=(_NPAD // rows,),
        in_specs=[
            pl.BlockSpec((rows, _H), lambda i: (i, 0)),
            pl.BlockSpec((rows, 1), lambda i: (i, 0)),
        ],
        out_specs=pl.BlockSpec((rows, _H), lambda i: (i, 0)),
        out_shape=jax.ShapeDtypeStruct((_NPAD, _H), jnp.float32),
    )(h2, dinv2d)


_BG = 50


def _tc_head(acc_g, m_g, dinv_g, bg2d, a1w, a1b2d, a2w, a2b2d, v1w, v1b2d,
             v2w, v2b2d, m1w, m1b2d, m2w, m2b2d, ew):
    def body(acc_ref, m_ref, dv_ref, bg_ref, a1_ref, a1b_ref, a2_ref, a2b_ref,
             v1_ref, v1b_ref, v2_ref, v2b_ref, m1_ref, m1b_ref, m2_ref,
             m2b_ref, ew_ref, o_ref):
        rows = _BG * _NPG
        accs = (acc_ref[0].astype(jnp.float32)
                + acc_ref[1].astype(jnp.float32)
                + m_ref[...])
        hg = jnp.maximum(dv_ref[...] * accs
                         + bg_ref[...].reshape(1, 1, _H), 0.0)
        hg2 = hg.reshape(rows, _H)
        t = jnp.maximum(jnp.dot(hg2, a1_ref[...],
                                preferred_element_type=jnp.float32)
                        + a1b_ref[...], 0.0)
        t = jnp.dot(t, a2_ref[...],
                    preferred_element_type=jnp.float32) + a2b_ref[...]
        t3 = t.reshape(_BG, _NPG, _H)
        proto = jnp.mean(t3, axis=1, keepdims=True)
        n1 = jnp.maximum(
            jnp.sqrt(jnp.sum(t3 * t3, axis=2, keepdims=True)), 1e-8)
        n2 = jnp.maximum(
            jnp.sqrt(jnp.sum(proto * proto, axis=2, keepdims=True)), 1e-8)
        sim = jnp.sum(t3 * proto, axis=2, keepdims=True) / (n1 * n2)
        att = (1.0 + sim) * 0.5
        mod = ew_ref[...] * att
        rs = jnp.sum(mod, axis=2, keepdims=True)
        rs = jnp.where(rs == 0.0, 1.0, rs)
        mod = mod / rs
        virt = lax.dot_general(mod, hg, (((1,), (1,)), ((0,), (0,))),
                               preferred_element_type=jnp.float32)
        virt2 = virt.reshape(_BG * _V, _H)
        virt2 = jnp.maximum(jnp.dot(virt2, v1_ref[...],
                                    preferred_element_type=jnp.float32)
                            + v1b_ref[...], 0.0)
        virt2 = jnp.dot(virt2, v2_ref[...],
                        preferred_element_type=jnp.float32) + v2b_ref[...]
        gf = jnp.mean(virt2.reshape(_BG, _V, _H), axis=1)
        og = jnp.maximum(jnp.dot(gf, m1_ref[...],
                                 preferred_element_type=jnp.float32)
                         + m1b_ref[...], 0.0)
        og = jnp.dot(og, m2_ref[...],
                     preferred_element_type=jnp.float32) + m2b_ref[...]
        o_ref[0] = og

    full = lambda *shape: pl.BlockSpec(shape, lambda g: (0,) * len(shape))
    out = pl.pallas_call(
        body,
        grid=(_G // _BG,),
        in_specs=[
            pl.BlockSpec((2, _BG, _NPG, _H), lambda g: (0, g, 0, 0)),
            pl.BlockSpec((_BG, _NPG, _H), lambda g: (g, 0, 0)),
            pl.BlockSpec((_BG, _NPG, 1), lambda g: (g, 0, 0)),
            full(1, _H),
            full(_H, _H), full(1, _H),
            full(_H, _H), full(1, _H),
            full(_H, _H), full(1, _H),
            full(_H, _H), full(1, _H),
            full(_H, _H), full(1, _H),
            full(_H, _OUT), full(1, _OUT),
            pl.BlockSpec((_BG, _NPG, _V), lambda g: (g, 0, 0)),
        ],
        out_specs=pl.BlockSpec((1, _BG, _OUT), lambda g: (g, 0, 0)),
        out_shape=jax.ShapeDtypeStruct((_G // _BG, _BG, _OUT), jnp.float32),
    )(acc_g, m_g, dinv_g, bg2d, a1w, a1b2d, a2w, a2b2d, v1w, v1b2d, v2w,
      v2b2d, m1w, m1b2d, m2w, m2b2d, ew)
    return out.reshape(_G, _OUT)


def kernel(x, edge_index, batch, W_emb, b_emb, W_gcn, b_gcn, A1_W, A1_b, A2_W,
           A2_b, V1_W, V1_b, V2_W, V2_b, M1_W, M1_b, M2_W, M2_b, edge_weights):
    del batch
    kb = 100
    src = edge_index[0].reshape(_E // kb, 1, kb)
    dst_flat = edge_index[1]
    dst = dst_flat.reshape(_E // kb, 1, kb)

    dinv = _deg_dinv(dst_flat)
    x_pad = jnp.pad(x, ((0, _NPAD - _N), (0, 0)))
    h2 = _tc_h2(x_pad, W_emb, b_emb.reshape(1, _H), W_gcn)
    m_s = _tc_scale(h2, dinv.reshape(_NPAD, 1))
    zrows = jnp.zeros((_NPAD, _H), jnp.float32)
    acc = _edge_scatter(m_s, src, dst, zrows)

    acc_g = acc.reshape(_NCORES, _NPAD, _H)[:, :_N].reshape(
        _NCORES, _G, _NPG, _H)
    m_g = m_s[:_N].reshape(_G, _NPG, _H)
    dinv_g = dinv[:_N].reshape(_G, _NPG, 1)
    return _tc_head(
        acc_g, m_g, dinv_g, b_gcn.reshape(1, _H),
        A1_W, A1_b.reshape(1, _H), A2_W, A2_b.reshape(1, _H),
        V1_W, V1_b.reshape(1, _H), V2_W, V2_b.reshape(1, _H),
        M1_W, M1_b.reshape(1, _H), M2_W, M2_b.reshape(1, _OUT),
        edge_weights)


# speedup vs baseline: 1.0420x; 1.0420x over previous
import functools

import jax
import jax.numpy as jnp
from jax import lax
from jax.experimental import pallas as pl
from jax.experimental.pallas import tpu as pltpu
from jax.experimental.pallas import tpu_sc as plsc

_N = 10000
_NPAD = 10240
_E = 320000
_H = 128
_G = 100
_NPG = 100
_V = 32
_OUT = 6

_LANES = 16
_NTILES = 16
_NCORES = 2


def _deg_hist(dst):
    ept = _E // (_NTILES * _NCORES)
    chunk = 2000
    colw = _NPAD // _NTILES
    mesh = plsc.VectorSubcoreMesh(core_axis_name="c", subcore_axis_name="s")

    @functools.partial(
        pl.kernel,
        out_type=jax.ShapeDtypeStruct((_NCORES, _NPAD), jnp.float32),
        mesh=mesh,
        compiler_params=pltpu.CompilerParams(needs_layout_passes=False),
        scratch_types=[
            pltpu.VMEM((chunk,), jnp.int32),
            pltpu.VMEM((_NPAD,), jnp.float32),
            pltpu.VMEM((colw,), jnp.float32),
            pltpu.VMEM_SHARED((_NTILES, _NPAD), jnp.float32),
        ],
    )
    def run(dst_hbm, hist_hbm, idx_v, hist_v, col_v, slab):
        c = lax.axis_index("c")
        s = lax.axis_index("s")
        zeros16 = jnp.zeros((_LANES,), jnp.float32)
        ones16 = jnp.ones((_LANES,), jnp.float32)

        def zero_hist(i, _):
            hist_v[pl.ds(i * _LANES, _LANES)] = zeros16
            return 0
        lax.fori_loop(0, _NPAD // _LANES, zero_hist, 0)

        base = (c * _NTILES + s) * ept

        def outer(ci, _):
            pltpu.sync_copy(dst_hbm.at[pl.ds(base + ci * chunk, chunk)],
                            idx_v)
            def inner(j, _):
                iv = idx_v[pl.ds(j * _LANES, _LANES)]
                plsc.addupdate_scatter(hist_v, [iv], ones16)
                return 0
            lax.fori_loop(0, chunk // _LANES, inner, 0)
            return 0
        lax.fori_loop(0, ept // chunk, outer, 0)
        pltpu.sync_copy(hist_v, slab.at[s])

        plsc.subcore_barrier()

        def init_col(i, _):
            col_v[pl.ds(i * _LANES, _LANES)] = zeros16
            return 0
        lax.fori_loop(0, colw // _LANES, init_col, 0)

        def add_row(k, _):
            pltpu.sync_copy(slab.at[k, pl.ds(s * colw, colw)],
                            hist_v.at[pl.ds(0, colw)])
            def accum(i, _):
                sl = pl.ds(i * _LANES, _LANES)
                col_v[sl] = col_v[sl] + hist_v[sl]
                return 0
            lax.fori_loop(0, colw // _LANES, accum, 0)
            return 0
        lax.fori_loop(0, _NTILES, add_row, 0)
        pltpu.sync_copy(col_v, hist_hbm.at[c, pl.ds(s * colw, colw)])

    return run(dst)


def _edge_scatter(m_s, src2d, dst2d, zrows):
    k_batch = src2d.shape[2]
    rows_per_tile = src2d.shape[0] // (_NTILES * _NCORES)
    accw = _NPAD // _NTILES
    nbuf = 3
    nouter = -(-(rows_per_tile + 1) // nbuf)
    mesh = plsc.VectorSubcoreMesh(core_axis_name="c", subcore_axis_name="s")

    scratch = [pltpu.VMEM_SHARED((_NPAD, _H), jnp.float32)]
    scratch += [pltpu.VMEM((k_batch, _H), jnp.float32) for _ in range(nbuf)]
    scratch += [pltpu.VMEM((1, k_batch), jnp.int32) for _ in range(2 * nbuf)]
    scratch += [pltpu.SemaphoreType.DMA for _ in range(3 * nbuf)]

    @functools.partial(
        pl.kernel,
        out_type=jax.ShapeDtypeStruct((_NCORES * _NPAD, _H), jnp.float32),
        mesh=mesh,
        compiler_params=pltpu.CompilerParams(needs_layout_passes=False),
        scratch_types=scratch,
    )
    def run(m_hbm, src_hbm, dst_hbm, z_hbm, out_hbm, acc, *rest):
        bufs = rest[:nbuf]
        si = rest[nbuf:2 * nbuf]
        di = rest[2 * nbuf:3 * nbuf]
        isem = rest[3 * nbuf:4 * nbuf]
        gsem = rest[4 * nbuf:5 * nbuf]
        ssem = rest[5 * nbuf:6 * nbuf]
        c = lax.axis_index("c")
        s = lax.axis_index("s")
        pltpu.sync_copy(z_hbm.at[pl.ds(s * accw, accw)],
                        acc.at[pl.ds(s * accw, accw)])
        base = (c * _NTILES + s) * rows_per_tile
        pltpu.async_copy(src_hbm.at[base], si[0], isem[0])
        pltpu.async_copy(dst_hbm.at[base], di[0], isem[0])
        plsc.subcore_barrier()

        def wait_idx(q):
            pltpu.make_async_copy(src_hbm.at[base], si[q],
                                  isem[q]).wait()
            pltpu.make_async_copy(dst_hbm.at[base], di[q],
                                  isem[q]).wait()

        def step(i, _):
            for j in range(nbuf):
                b = i * nbuf + j
                jn = (j + 1) % nbuf
                jp = (j - 1) % nbuf
                @pl.when(b + 1 < rows_per_tile)
                def _():
                    @pl.when(b >= 2)
                    def _():
                        pltpu.make_async_copy(
                            bufs[jn], acc.at[di[jn].at[0]], ssem[jn]).wait()
                    pltpu.async_copy(src_hbm.at[base + b + 1], si[jn],
                                     isem[jn])
                    pltpu.async_copy(dst_hbm.at[base + b + 1], di[jn],
                                     isem[jn])
                @pl.when(b < rows_per_tile)
                def _():
                    wait_idx(j)
                    pltpu.async_copy(m_hbm.at[si[j].at[0]], bufs[j], gsem[j])
                @pl.when(jnp.logical_and(b >= 1, b - 1 < rows_per_tile))
                def _():
                    pltpu.make_async_copy(
                        m_hbm.at[si[jp].at[0]], bufs[jp], gsem[jp]).wait()
                    pltpu.async_copy(bufs[jp], acc.at[di[jp].at[0]], ssem[jp],
                                     add=True)
            return 0
        lax.fori_loop(0, nouter, step, 0)

        for j in range(nbuf):
            pltpu.make_async_copy(bufs[j], acc.at[di[j].at[0]],
                                  ssem[j]).wait()

        plsc.subcore_barrier()
        pltpu.sync_copy(acc.at[pl.ds(s * accw, accw)],
                        out_hbm.at[pl.ds(c * _NPAD + s * accw, accw)])

    return run(m_s, src2d, dst2d, zrows)


def _tc_h2(x_pad, w_emb, b_emb2d, w_gcn):
    rows = 1280

    def body(x_ref, we_ref, be_ref, wg_ref, o_ref):
        h = jnp.dot(x_ref[...], we_ref[...],
                    preferred_element_type=jnp.float32) + be_ref[...]
        o_ref[...] = jnp.dot(h, wg_ref[...],
                             preferred_element_type=jnp.float32)

    return pl.pallas_call(
        body,
        grid=(_NPAD // rows,),
        in_specs=[
            pl.BlockSpec((rows, _H), lambda i: (i, 0)),
            pl.BlockSpec((_H, _H), lambda i: (0, 0)),
            pl.BlockSpec((1, _H), lambda i: (0, 0)),
            pl.BlockSpec((_H, _H), lambda i: (0, 0)),
        ],
        out_specs=pl.BlockSpec((rows, _H), lambda i: (i, 0)),
        out_shape=jax.ShapeDtypeStruct((_NPAD, _H), jnp.float32),
    )(x_pad, w_emb, b_emb2d, w_gcn)


def _tc_scale(h2, hist):
    rows = 5120

    def body(h_ref, hist_ref, o_ref, dv_ref):
        ones2 = jnp.ones((2, 1), jnp.float32)
        deg = lax.dot_general(hist_ref[...], ones2, (((0,), (0,)), ((), ())),
                              preferred_element_type=jnp.float32) + 1.0
        dv = 1.0 / jnp.sqrt(deg)
        o_ref[...] = h_ref[...] * dv
        dv_ref[...] = dv

    return pl.pallas_call(
        body,
        grid=(_NPAD // rows,),
        in_specs=[
            pl.BlockSpec((rows, _H), lambda i: (i, 0)),
            pl.BlockSpec((2, rows), lambda i: (0, i)),
        ],
        out_specs=[pl.BlockSpec((rows, _H), lambda i: (i, 0)),
                   pl.BlockSpec((rows, 1), lambda i: (i, 0))],
        out_shape=[jax.ShapeDtypeStruct((_NPAD, _H), jnp.float32),
                   jax.ShapeDtypeStruct((_NPAD, 1), jnp.float32)],
    )(h2, hist)


_BG = 25


def _tc_head(acc_g, m_g, dinv_g, bg2d, a1w, a1b2d, a2w, a2b2d, v1w, v1b2d,
             v2w, v2b2d, m1w, m1b2d, m2w, m2b2d, ew):
    def body(acc_ref, m_ref, dv_ref, bg_ref, a1_ref, a1b_ref, a2_ref, a2b_ref,
             v1_ref, v1b_ref, v2_ref, v2b_ref, m1_ref, m1b_ref, m2_ref,
             m2b_ref, ew_ref, o_ref):
        rows = _BG * _NPG
        accs = (acc_ref[0].astype(jnp.float32)
                + acc_ref[1].astype(jnp.float32)
                + m_ref[...])
        hg = jnp.maximum(dv_ref[...] * accs
                         + bg_ref[...].reshape(1, 1, _H), 0.0)
        hg2 = hg.reshape(rows, _H)
        t = jnp.maximum(jnp.dot(hg2, a1_ref[...],
                                preferred_element_type=jnp.float32)
                        + a1b_ref[...], 0.0)
        t = jnp.dot(t, a2_ref[...],
                    preferred_element_type=jnp.float32) + a2b_ref[...]
        t3 = t.reshape(_BG, _NPG, _H)
        proto = jnp.mean(t3, axis=1, keepdims=True)
        n1 = jnp.maximum(
            jnp.sqrt(jnp.sum(t3 * t3, axis=2, keepdims=True)), 1e-8)
        n2 = jnp.maximum(
            jnp.sqrt(jnp.sum(proto * proto, axis=2, keepdims=True)), 1e-8)
        sim = jnp.sum(t3 * proto, axis=2, keepdims=True) / (n1 * n2)
        att = (1.0 + sim) * 0.5
        mod = ew_ref[...] * att
        rs = jnp.sum(mod, axis=2, keepdims=True)
        rs = jnp.where(rs == 0.0, 1.0, rs)
        mod = mod / rs
        virt = lax.dot_general(mod, hg, (((1,), (1,)), ((0,), (0,))),
                               preferred_element_type=jnp.float32)
        virt2 = virt.reshape(_BG * _V, _H)
        virt2 = jnp.maximum(jnp.dot(virt2, v1_ref[...],
                                    preferred_element_type=jnp.float32)
                            + v1b_ref[...], 0.0)
        virt2 = jnp.dot(virt2, v2_ref[...],
                        preferred_element_type=jnp.float32) + v2b_ref[...]
        gf = jnp.mean(virt2.reshape(_BG, _V, _H), axis=1)
        og = jnp.maximum(jnp.dot(gf, m1_ref[...],
                                 preferred_element_type=jnp.float32)
                         + m1b_ref[...], 0.0)
        og = jnp.dot(og, m2_ref[...],
                     preferred_element_type=jnp.float32) + m2b_ref[...]
        o_ref[0] = og

    full = lambda *shape: pl.BlockSpec(shape, lambda g: (0,) * len(shape))
    out = pl.pallas_call(
        body,
        grid=(_G // _BG,),
        in_specs=[
            pl.BlockSpec((2, _BG, _NPG, _H), lambda g: (0, g, 0, 0)),
            pl.BlockSpec((_BG, _NPG, _H), lambda g: (g, 0, 0)),
            pl.BlockSpec((_BG, _NPG, 1), lambda g: (g, 0, 0)),
            full(1, _H),
            full(_H, _H), full(1, _H),
            full(_H, _H), full(1, _H),
            full(_H, _H), full(1, _H),
            full(_H, _H), full(1, _H),
            full(_H, _H), full(1, _H),
            full(_H, _OUT), full(1, _OUT),
            pl.BlockSpec((_BG, _NPG, _V), lambda g: (g, 0, 0)),
        ],
        out_specs=pl.BlockSpec((1, _BG, _OUT), lambda g: (g, 0, 0)),
        out_shape=jax.ShapeDtypeStruct((_G // _BG, _BG, _OUT), jnp.float32),
    )(acc_g, m_g, dinv_g, bg2d, a1w, a1b2d, a2w, a2b2d, v1w, v1b2d, v2w,
      v2b2d, m1w, m1b2d, m2w, m2b2d, ew)
    return out.reshape(_G, _OUT)


def kernel(x, edge_index, batch, W_emb, b_emb, W_gcn, b_gcn, A1_W, A1_b, A2_W,
           A2_b, V1_W, V1_b, V2_W, V2_b, M1_W, M1_b, M2_W, M2_b, edge_weights):
    del batch
    kb = 100
    src = edge_index[0].reshape(_E // kb, 1, kb)
    dst_flat = edge_index[1]
    dst = dst_flat.reshape(_E // kb, 1, kb)

    hist = _deg_hist(dst_flat)
    x_pad = jnp.pad(x, ((0, _NPAD - _N), (0, 0)))
    h2 = _tc_h2(x_pad, W_emb, b_emb.reshape(1, _H), W_gcn)
    m_s, dinv2d = _tc_scale(h2, hist)
    zrows = jnp.zeros((_NPAD, _H), jnp.float32)
    acc = _edge_scatter(m_s, src, dst, zrows)

    acc_g = acc.reshape(_NCORES, _NPAD, _H)[:, :_N].reshape(
        _NCORES, _G, _NPG, _H)
    m_g = m_s[:_N].reshape(_G, _NPG, _H)
    dinv_g = dinv2d[:_N].reshape(_G, _NPG, 1)
    return _tc_head(
        acc_g, m_g, dinv_g, b_gcn.reshape(1, _H),
        A1_W, A1_b.reshape(1, _H), A2_W, A2_b.reshape(1, _H),
        V1_W, V1_b.reshape(1, _H), V2_W, V2_b.reshape(1, _H),
        M1_W, M1_b.reshape(1, _H), M2_W, M2_b.reshape(1, _OUT),
        edge_weights)
